# baseline (speedup 1.0000x reference)
import jax
import jax.numpy as jnp
from jax.experimental import pallas as pl

K_CODES = 1024
LATENT_DIM = 256


def _vq_body(z_ref, w2_ref, zn_ref, wn_ref, iota_ref, zq_ref, idx_ref, prob_ref):
    zt = z_ref[0]
    W2 = w2_ref[...]
    k = W2.shape[0]
    mm2 = jax.lax.dot_general(
        zt, W2, (((0,), (1,)), ((), ())),
        preferred_element_type=jnp.float32,
    )
    zn_col = zn_ref[0]
    wn_row = wn_ref[...]
    dist = (zn_col + wn_row) - mm2
    rowmin = jnp.min(dist, axis=1, keepdims=True)
    e = jnp.exp(rowmin - dist)
    prob_ref[0] = e * (1.0 / jnp.sum(e, axis=1, keepdims=True))
    iota_row = iota_ref[...]
    masked = jnp.where(dist == rowmin, iota_row, float(k))
    idx_col = jnp.min(masked, axis=1, keepdims=True)
    idx_ref[0] = idx_col.astype(jnp.int32)
    sel = jnp.where(iota_row == idx_col, 0.5, 0.0)
    zq_ref[0] = jax.lax.dot_general(
        W2, sel, (((0,), (1,)), ((), ())),
        preferred_element_type=jnp.float32,
    )


@jax.jit
def kernel(z, W):
    b, d, h, w = z.shape
    n = h * w
    z_r = z.reshape(b, d, n)
    zn2 = jnp.sum(z_r ** 2, axis=1)[..., None]
    wn2 = jnp.sum(W ** 2, axis=1)
    w2 = W + W
    iota_row = jnp.arange(K_CODES, dtype=jnp.float32).reshape(1, K_CODES)
    zq, idx, prob = pl.pallas_call(
        _vq_body,
        grid=(b,),
        in_specs=[
            pl.BlockSpec((1, d, n), lambda i: (i, 0, 0)),
            pl.BlockSpec((K_CODES, d), lambda i: (0, 0)),
            pl.BlockSpec((1, n, 1), lambda i: (i, 0, 0)),
            pl.BlockSpec((1, K_CODES), lambda i: (0, 0)),
            pl.BlockSpec((1, K_CODES), lambda i: (0, 0)),
        ],
        out_specs=[
            pl.BlockSpec((1, d, n), lambda i: (i, 0, 0)),
            pl.BlockSpec((1, n, 1), lambda i: (i, 0, 0)),
            pl.BlockSpec((1, n, K_CODES), lambda i: (i, 0, 0)),
        ],
        out_shape=[
            jax.ShapeDtypeStruct((b, d, n), jnp.float32),
            jax.ShapeDtypeStruct((b, n, 1), jnp.int32),
            jax.ShapeDtypeStruct((b, n, K_CODES), jnp.float32),
        ],
    )(z_r, w2, zn2, wn2.reshape(1, K_CODES), iota_row)
    return (
        zq.reshape(b, d, h, w),
        idx.reshape(b * n),
        prob.reshape(b * n, K_CODES),
    )
